# baseline (speedup 1.0000x reference)
import jax
import jax.numpy as jnp
from jax import lax
from jax.experimental import pallas as pl
from jax.experimental.pallas import tpu as pltpu

N = 4096
IN_FEAT = 512
HID_FEAT = 256
OUT_FEAT = 64
CT = 20

RE = 256
NE = N // RE
RA = 256
NA = N // RA

S_EF = NE
S_LS = 2 * NE
S_ZS = S_LS + NA
S_LF = S_ZS + NA
S_ZF = S_LF + NA
S_END = S_ZF + NA


def _encode_kernel(fs_ref, ff_ref, as_ref, af_ref, w1_ref, w2_ref,
                   zs_ref, zf_ref,
                   gs_scr, gf_scr, q_scr, ab_scr):
    i = pl.program_id(0)

    @pl.when(i < S_EF)
    def _embed_s():
        h = jnp.dot(fs_ref[...], w1_ref[...], preferred_element_type=jnp.float32)
        gs_scr[pl.ds(i * RE, RE), :] = jnp.dot(
            h, w2_ref[...], preferred_element_type=jnp.float32)

    @pl.when((i >= S_EF) & (i < S_LS))
    def _embed_f():
        h = jnp.dot(ff_ref[...], w1_ref[...], preferred_element_type=jnp.float32)
        gf_scr[pl.ds((i - S_EF) * RE, RE), :] = jnp.dot(
            h, w2_ref[...], preferred_element_type=jnp.float32)

    @pl.when((i >= S_LS) & (i < S_ZS))
    def _load_s():
        blk = as_ref[...]
        k = i - S_LS
        q_scr[pl.ds(k * RA, RA), :] = jnp.dot(
            blk, gs_scr[...], preferred_element_type=jnp.float32
        ).astype(jnp.bfloat16)
        ab_scr[pl.ds(k * RA, RA), :] = blk.astype(jnp.bfloat16)

    @pl.when((i >= S_ZS) & (i < S_LF))
    def _z_s():
        k = i - S_ZS
        zs_ref[...] = jnp.dot(ab_scr[pl.ds(k * RA, RA), :], q_scr[...],
                              preferred_element_type=jnp.float32)

    @pl.when((i >= S_LF) & (i < S_ZF))
    def _load_f():
        blk = af_ref[...]
        k = i - S_LF
        q_scr[pl.ds(k * RA, RA), :] = jnp.dot(
            blk, gf_scr[...], preferred_element_type=jnp.float32
        ).astype(jnp.bfloat16)
        ab_scr[pl.ds(k * RA, RA), :] = blk.astype(jnp.bfloat16)

    @pl.when(i >= S_ZF)
    def _z_f():
        k = i - S_ZF
        zf_ref[...] = jnp.dot(ab_scr[pl.ds(k * RA, RA), :], q_scr[...],
                              preferred_element_type=jnp.float32)


def _fs_idx(i):
    return (jnp.minimum(i, NE - 1), 0)


def _ff_idx(i):
    return (jnp.clip(i - S_EF, 0, NE - 1), 0)


def _as_idx(i):
    return (jnp.clip(i - S_LS, 0, NA - 1), 0)


def _af_idx(i):
    return (jnp.clip(i - S_LF, 0, NA - 1), 0)


def _zs_idx(i):
    return (jnp.clip(i - S_ZS, 0, NA - 1), 0)


def _zf_idx(i):
    return (jnp.clip(i - S_ZF, 0, NA - 1), 0)


def _pin(i):
    return (0, 0)


def _encode(feat_s, feat_f, adj_s, adj_f, W1, W2):
    f32 = jnp.float32
    bf16 = jnp.bfloat16
    return pl.pallas_call(
        _encode_kernel,
        grid=(S_END,),
        in_specs=[
            pl.BlockSpec((RE, IN_FEAT), _fs_idx),
            pl.BlockSpec((RE, IN_FEAT), _ff_idx),
            pl.BlockSpec((RA, N), _as_idx),
            pl.BlockSpec((RA, N), _af_idx),
            pl.BlockSpec((IN_FEAT, HID_FEAT), _pin),
            pl.BlockSpec((HID_FEAT, OUT_FEAT), _pin),
        ],
        out_specs=[
            pl.BlockSpec((RA, OUT_FEAT), _zs_idx),
            pl.BlockSpec((RA, OUT_FEAT), _zf_idx),
        ],
        out_shape=[
            jax.ShapeDtypeStruct((N, OUT_FEAT), f32),
            jax.ShapeDtypeStruct((N, OUT_FEAT), f32),
        ],
        scratch_shapes=[
            pltpu.VMEM((N, OUT_FEAT), f32),
            pltpu.VMEM((N, OUT_FEAT), f32),
            pltpu.VMEM((N, OUT_FEAT), bf16),
            pltpu.VMEM((N, N), bf16),
        ],
        compiler_params=pltpu.CompilerParams(
            dimension_semantics=("arbitrary",),
            vmem_limit_bytes=63 * 1024 * 1024,
        ),
    )(feat_s, feat_f, adj_s, adj_f, W1, W2)


def _decode_one(zb, z_all, wp, bp, wr, br, arec_ref, pred_ref, rec_ref):
    prod = lax.dot_general(zb, z_all, (((1,), (1,)), ((), ())),
                           preferred_element_type=jnp.float32)
    arec_ref[...] = jax.nn.sigmoid(prod)
    logits = jnp.dot(zb, wp, preferred_element_type=jnp.float32) + bp
    pred_ref[...] = jax.nn.softmax(logits, axis=-1)
    rec_ref[...] = jnp.dot(zb, wr, preferred_element_type=jnp.float32) + br


def _decode_kernel(zbs_ref, zbf_ref, zs_ref, zf_ref, wp_ref, bp_ref, wr_ref,
                   br_ref, arecs_ref, preds_ref, recs_ref,
                   arecf_ref, predf_ref, recf_ref):
    wp = wp_ref[...]
    bp = bp_ref[...]
    wr = wr_ref[...]
    br = br_ref[...]
    _decode_one(zbs_ref[...], zs_ref[...], wp, bp, wr, br,
                arecs_ref, preds_ref, recs_ref)
    _decode_one(zbf_ref[...], zf_ref[...], wp, bp, wr, br,
                arecf_ref, predf_ref, recf_ref)


RD = 512
ND = N // RD


def _blk(i):
    return (i, 0)


def _decode(z_s, z_f, Wp, bp2, Wr, br2):
    f32 = jnp.float32
    return pl.pallas_call(
        _decode_kernel,
        grid=(ND,),
        in_specs=[
            pl.BlockSpec((RD, OUT_FEAT), _blk),
            pl.BlockSpec((RD, OUT_FEAT), _blk),
            pl.BlockSpec((N, OUT_FEAT), _pin),
            pl.BlockSpec((N, OUT_FEAT), _pin),
            pl.BlockSpec((OUT_FEAT, CT), _pin),
            pl.BlockSpec((1, CT), _pin),
            pl.BlockSpec((OUT_FEAT, IN_FEAT), _pin),
            pl.BlockSpec((1, IN_FEAT), _pin),
        ],
        out_specs=[
            pl.BlockSpec((RD, N), _blk),
            pl.BlockSpec((RD, CT), _blk),
            pl.BlockSpec((RD, IN_FEAT), _blk),
            pl.BlockSpec((RD, N), _blk),
            pl.BlockSpec((RD, CT), _blk),
            pl.BlockSpec((RD, IN_FEAT), _blk),
        ],
        out_shape=[
            jax.ShapeDtypeStruct((N, N), f32),
            jax.ShapeDtypeStruct((N, CT), f32),
            jax.ShapeDtypeStruct((N, IN_FEAT), f32),
            jax.ShapeDtypeStruct((N, N), f32),
            jax.ShapeDtypeStruct((N, CT), f32),
            jax.ShapeDtypeStruct((N, IN_FEAT), f32),
        ],
        compiler_params=pltpu.CompilerParams(
            dimension_semantics=("arbitrary",),
            vmem_limit_bytes=63 * 1024 * 1024,
        ),
    )(z_s, z_f, z_s, z_f, Wp, bp2, Wr, br2)


def kernel(features, features_sc, adj_spatial, adj_feature, W1, W2, Wp, bp, Wr, br):
    bp2 = bp.reshape(1, CT)
    br2 = br.reshape(1, IN_FEAT)

    z_s, z_f = _encode(features, features_sc, adj_spatial, adj_feature, W1, W2)
    arec_s, pred_s, rec_s, arec_f, pred_f, rec_f = _decode(
        z_s, z_f, Wp, bp2, Wr, br2)

    return (z_s, z_f, rec_s, rec_f, arec_s, arec_f, pred_s, pred_f)


# speedup vs baseline: 1.1364x; 1.1364x over previous
import jax
import jax.numpy as jnp
from jax import lax
from jax.experimental import pallas as pl
from jax.experimental.pallas import tpu as pltpu

N = 4096
IN_FEAT = 512
HID_FEAT = 256
OUT_FEAT = 64
CT = 20

RB = 512
NB = N // RB

S_L = NB
S_Z = 2 * NB
S_END = 3 * NB


def _encode_kernel(f_ref, a_ref, w1_ref, w2_ref, z_ref,
                   g_scr, q_scr, ab_scr):
    i = pl.program_id(0)

    @pl.when(i < S_L)
    def _embed():
        h = jnp.dot(f_ref[...], w1_ref[...], preferred_element_type=jnp.float32)
        g_scr[pl.ds(i * RB, RB), :] = jnp.dot(
            h, w2_ref[...], preferred_element_type=jnp.float32)

    @pl.when((i >= S_L) & (i < S_Z))
    def _load():
        blk = a_ref[...]
        k = i - S_L
        q_scr[pl.ds(k * RB, RB), :] = jnp.dot(
            blk, g_scr[...], preferred_element_type=jnp.float32
        ).astype(jnp.bfloat16)
        ab_scr[pl.ds(k * RB, RB), :] = blk.astype(jnp.bfloat16)

    @pl.when(i >= S_Z)
    def _z():
        k = i - S_Z
        z_ref[...] = jnp.dot(ab_scr[pl.ds(k * RB, RB), :], q_scr[...],
                             preferred_element_type=jnp.float32)


def _f_idx(i):
    return (jnp.minimum(i, NB - 1), 0)


def _a_idx(i):
    return (jnp.clip(i - S_L, 0, NB - 1), 0)


def _z_idx(i):
    return (jnp.clip(i - S_Z, 0, NB - 1), 0)


def _pin(i):
    return (0, 0)


def _encode(feat, adj, W1, W2):
    f32 = jnp.float32
    bf16 = jnp.bfloat16
    return pl.pallas_call(
        _encode_kernel,
        grid=(S_END,),
        in_specs=[
            pl.BlockSpec((RB, IN_FEAT), _f_idx),
            pl.BlockSpec((RB, N), _a_idx),
            pl.BlockSpec((IN_FEAT, HID_FEAT), _pin),
            pl.BlockSpec((HID_FEAT, OUT_FEAT), _pin),
        ],
        out_specs=pl.BlockSpec((RB, OUT_FEAT), _z_idx),
        out_shape=jax.ShapeDtypeStruct((N, OUT_FEAT), f32),
        scratch_shapes=[
            pltpu.VMEM((N, OUT_FEAT), f32),
            pltpu.VMEM((N, OUT_FEAT), bf16),
            pltpu.VMEM((N, N), bf16),
        ],
        compiler_params=pltpu.CompilerParams(
            dimension_semantics=("arbitrary",),
            vmem_limit_bytes=63 * 1024 * 1024,
        ),
    )(feat, adj, W1, W2)


def _decode_one(zb, z_all, wp, bp, wr, br, arec_ref, pred_ref, rec_ref):
    prod = lax.dot_general(zb, z_all, (((1,), (1,)), ((), ())),
                           preferred_element_type=jnp.float32)
    arec_ref[...] = jax.nn.sigmoid(prod)
    logits = jnp.dot(zb, wp, preferred_element_type=jnp.float32) + bp
    pred_ref[...] = jax.nn.softmax(logits, axis=-1)
    rec_ref[...] = jnp.dot(zb, wr, preferred_element_type=jnp.float32) + br


def _decode_kernel(zbs_ref, zbf_ref, zs_ref, zf_ref, wp_ref, bp_ref, wr_ref,
                   br_ref, arecs_ref, preds_ref, recs_ref,
                   arecf_ref, predf_ref, recf_ref):
    wp = wp_ref[...]
    bp = bp_ref[...]
    wr = wr_ref[...]
    br = br_ref[...]
    _decode_one(zbs_ref[...], zs_ref[...], wp, bp, wr, br,
                arecs_ref, preds_ref, recs_ref)
    _decode_one(zbf_ref[...], zf_ref[...], wp, bp, wr, br,
                arecf_ref, predf_ref, recf_ref)


def _blk(i):
    return (i, 0)


def _decode(z_s, z_f, Wp, bp2, Wr, br2):
    f32 = jnp.float32
    return pl.pallas_call(
        _decode_kernel,
        grid=(NB,),
        in_specs=[
            pl.BlockSpec((RB, OUT_FEAT), _blk),
            pl.BlockSpec((RB, OUT_FEAT), _blk),
            pl.BlockSpec((N, OUT_FEAT), _pin),
            pl.BlockSpec((N, OUT_FEAT), _pin),
            pl.BlockSpec((OUT_FEAT, CT), _pin),
            pl.BlockSpec((1, CT), _pin),
            pl.BlockSpec((OUT_FEAT, IN_FEAT), _pin),
            pl.BlockSpec((1, IN_FEAT), _pin),
        ],
        out_specs=[
            pl.BlockSpec((RB, N), _blk),
            pl.BlockSpec((RB, CT), _blk),
            pl.BlockSpec((RB, IN_FEAT), _blk),
            pl.BlockSpec((RB, N), _blk),
            pl.BlockSpec((RB, CT), _blk),
            pl.BlockSpec((RB, IN_FEAT), _blk),
        ],
        out_shape=[
            jax.ShapeDtypeStruct((N, N), f32),
            jax.ShapeDtypeStruct((N, CT), f32),
            jax.ShapeDtypeStruct((N, IN_FEAT), f32),
            jax.ShapeDtypeStruct((N, N), f32),
            jax.ShapeDtypeStruct((N, CT), f32),
            jax.ShapeDtypeStruct((N, IN_FEAT), f32),
        ],
        compiler_params=pltpu.CompilerParams(
            dimension_semantics=("arbitrary",),
            vmem_limit_bytes=63 * 1024 * 1024,
        ),
    )(z_s, z_f, z_s, z_f, Wp, bp2, Wr, br2)


def kernel(features, features_sc, adj_spatial, adj_feature, W1, W2, Wp, bp, Wr, br):
    bp2 = bp.reshape(1, CT)
    br2 = br.reshape(1, IN_FEAT)

    z_s = _encode(features, adj_spatial, W1, W2)
    z_f = _encode(features_sc, adj_feature, W1, W2)
    arec_s, pred_s, rec_s, arec_f, pred_f, rec_f = _decode(
        z_s, z_f, Wp, bp2, Wr, br2)

    return (z_s, z_f, rec_s, rec_f, arec_s, arec_f, pred_s, pred_f)


# speedup vs baseline: 1.1398x; 1.0029x over previous
import jax
import jax.numpy as jnp
from jax import lax
from jax.experimental import pallas as pl
from jax.experimental.pallas import tpu as pltpu

N = 4096
IN_FEAT = 512
HID_FEAT = 256
OUT_FEAT = 64
CT = 20

RB = 512
NB = N // RB

S_L = NB
S_Z = 2 * NB
S_END = 3 * NB


NH = N // 2


def _encode_kernel(f_ref, a1_ref, a2_ref, w1_ref, w2_ref, z_ref,
                   g_scr, q_scr, ab_scr):
    i = pl.program_id(0)

    @pl.when(i < S_L)
    def _embed():
        h = jnp.dot(f_ref[...], w1_ref[...], preferred_element_type=jnp.float32)
        g_scr[pl.ds(i * RB, RB), :] = jnp.dot(
            h, w2_ref[...], preferred_element_type=jnp.float32)

    @pl.when((i >= S_L) & (i < S_Z))
    def _load():
        b1 = a1_ref[...]
        b2 = a2_ref[...]
        k = i - S_L
        q = jnp.dot(b1, g_scr[pl.ds(0, NH), :],
                    preferred_element_type=jnp.float32)
        q += jnp.dot(b2, g_scr[pl.ds(NH, NH), :],
                     preferred_element_type=jnp.float32)
        q_scr[pl.ds(k * RB, RB), :] = q.astype(jnp.bfloat16)
        ab_scr[pl.ds(k * RB, RB), 0:NH] = b1.astype(jnp.bfloat16)
        ab_scr[pl.ds(k * RB, RB), NH:N] = b2.astype(jnp.bfloat16)

    @pl.when(i >= S_Z)
    def _z():
        k = i - S_Z
        z_ref[...] = jnp.dot(ab_scr[pl.ds(k * RB, RB), :], q_scr[...],
                             preferred_element_type=jnp.float32)


def _f_idx(i):
    return (jnp.minimum(i, NB - 1), 0)


def _a1_idx(i):
    return (jnp.clip(i - S_L, 0, NB - 1), 0)


def _a2_idx(i):
    return (jnp.clip(i - S_L, 0, NB - 1), 1)


def _z_idx(i):
    return (jnp.clip(i - S_Z, 0, NB - 1), 0)


def _pin(i):
    return (0, 0)


def _encode(feat, adj, W1, W2):
    f32 = jnp.float32
    bf16 = jnp.bfloat16
    return pl.pallas_call(
        _encode_kernel,
        grid=(S_END,),
        in_specs=[
            pl.BlockSpec((RB, IN_FEAT), _f_idx),
            pl.BlockSpec((RB, NH), _a1_idx),
            pl.BlockSpec((RB, NH), _a2_idx),
            pl.BlockSpec((IN_FEAT, HID_FEAT), _pin),
            pl.BlockSpec((HID_FEAT, OUT_FEAT), _pin),
        ],
        out_specs=pl.BlockSpec((RB, OUT_FEAT), _z_idx),
        out_shape=jax.ShapeDtypeStruct((N, OUT_FEAT), f32),
        scratch_shapes=[
            pltpu.VMEM((N, OUT_FEAT), f32),
            pltpu.VMEM((N, OUT_FEAT), bf16),
            pltpu.VMEM((N, N), bf16),
        ],
        compiler_params=pltpu.CompilerParams(
            dimension_semantics=("arbitrary",),
            vmem_limit_bytes=63 * 1024 * 1024,
        ),
    )(feat, adj, adj, W1, W2)


def _decode_one(zb, z_all, wp, bp, wr, br, arec_ref, pred_ref, rec_ref):
    prod = lax.dot_general(zb, z_all, (((1,), (1,)), ((), ())),
                           preferred_element_type=jnp.float32)
    arec_ref[...] = jax.nn.sigmoid(prod)
    logits = jnp.dot(zb, wp, preferred_element_type=jnp.float32) + bp
    pred_ref[...] = jax.nn.softmax(logits, axis=-1)
    rec_ref[...] = jnp.dot(zb, wr, preferred_element_type=jnp.float32) + br


def _decode_kernel(zbs_ref, zbf_ref, zs_ref, zf_ref, wp_ref, bp_ref, wr_ref,
                   br_ref, arecs_ref, preds_ref, recs_ref,
                   arecf_ref, predf_ref, recf_ref):
    wp = wp_ref[...]
    bp = bp_ref[...]
    wr = wr_ref[...]
    br = br_ref[...]
    _decode_one(zbs_ref[...], zs_ref[...], wp, bp, wr, br,
                arecs_ref, preds_ref, recs_ref)
    _decode_one(zbf_ref[...], zf_ref[...], wp, bp, wr, br,
                arecf_ref, predf_ref, recf_ref)


def _blk(i):
    return (i, 0)


def _decode(z_s, z_f, Wp, bp2, Wr, br2):
    f32 = jnp.float32
    return pl.pallas_call(
        _decode_kernel,
        grid=(NB,),
        in_specs=[
            pl.BlockSpec((RB, OUT_FEAT), _blk),
            pl.BlockSpec((RB, OUT_FEAT), _blk),
            pl.BlockSpec((N, OUT_FEAT), _pin),
            pl.BlockSpec((N, OUT_FEAT), _pin),
            pl.BlockSpec((OUT_FEAT, CT), _pin),
            pl.BlockSpec((1, CT), _pin),
            pl.BlockSpec((OUT_FEAT, IN_FEAT), _pin),
            pl.BlockSpec((1, IN_FEAT), _pin),
        ],
        out_specs=[
            pl.BlockSpec((RB, N), _blk),
            pl.BlockSpec((RB, CT), _blk),
            pl.BlockSpec((RB, IN_FEAT), _blk),
            pl.BlockSpec((RB, N), _blk),
            pl.BlockSpec((RB, CT), _blk),
            pl.BlockSpec((RB, IN_FEAT), _blk),
        ],
        out_shape=[
            jax.ShapeDtypeStruct((N, N), f32),
            jax.ShapeDtypeStruct((N, CT), f32),
            jax.ShapeDtypeStruct((N, IN_FEAT), f32),
            jax.ShapeDtypeStruct((N, N), f32),
            jax.ShapeDtypeStruct((N, CT), f32),
            jax.ShapeDtypeStruct((N, IN_FEAT), f32),
        ],
        compiler_params=pltpu.CompilerParams(
            dimension_semantics=("arbitrary",),
            vmem_limit_bytes=63 * 1024 * 1024,
        ),
    )(z_s, z_f, z_s, z_f, Wp, bp2, Wr, br2)


def kernel(features, features_sc, adj_spatial, adj_feature, W1, W2, Wp, bp, Wr, br):
    bp2 = bp.reshape(1, CT)
    br2 = br.reshape(1, IN_FEAT)

    z_s = _encode(features, adj_spatial, W1, W2)
    z_f = _encode(features_sc, adj_feature, W1, W2)
    arec_s, pred_s, rec_s, arec_f, pred_f, rec_f = _decode(
        z_s, z_f, Wp, bp2, Wr, br2)

    return (z_s, z_f, rec_s, rec_f, arec_s, arec_f, pred_s, pred_f)


# speedup vs baseline: 1.1812x; 1.0364x over previous
import jax
import jax.numpy as jnp
from jax import lax
from jax.experimental import pallas as pl
from jax.experimental.pallas import tpu as pltpu

N = 4096
IN_FEAT = 512
HID_FEAT = 256
OUT_FEAT = 64
CT = 20

RB = 512
NB = N // RB

S_Z = NB
S_END = 2 * NB


def _encode_kernel(f_ref, a_ref, w1_ref, w2_ref, z_ref, q_scr, ab_scr):
    i = pl.program_id(0)

    @pl.when(i < S_Z)
    def _load():
        h = jnp.dot(f_ref[...], w1_ref[...], preferred_element_type=jnp.float32)
        g = jnp.dot(h, w2_ref[...], preferred_element_type=jnp.float32)
        blk = a_ref[...]
        p = jnp.dot(blk, g, preferred_element_type=jnp.float32)

        @pl.when(i == 0)
        def _init():
            q_scr[...] = p

        @pl.when(i > 0)
        def _acc():
            q_scr[...] += p

        ab_scr[:, pl.ds(i * RB, RB)] = blk.astype(jnp.bfloat16)

    @pl.when(i >= S_Z)
    def _z():
        k = i - S_Z
        z_ref[...] = jnp.dot(ab_scr[pl.ds(k * RB, RB), :],
                             q_scr[...].astype(jnp.bfloat16),
                             preferred_element_type=jnp.float32)


def _f_idx(i):
    return (jnp.minimum(i, NB - 1), 0)


def _a_idx(i):
    return (0, jnp.minimum(i, NB - 1))


def _z_idx(i):
    return (jnp.clip(i - S_Z, 0, NB - 1), 0)


def _pin(i):
    return (0, 0)


def _encode(feat, adj, W1, W2):
    f32 = jnp.float32
    bf16 = jnp.bfloat16
    return pl.pallas_call(
        _encode_kernel,
        grid=(S_END,),
        in_specs=[
            pl.BlockSpec((RB, IN_FEAT), _f_idx),
            pl.BlockSpec((N, RB), _a_idx),
            pl.BlockSpec((IN_FEAT, HID_FEAT), _pin),
            pl.BlockSpec((HID_FEAT, OUT_FEAT), _pin),
        ],
        out_specs=pl.BlockSpec((RB, OUT_FEAT), _z_idx),
        out_shape=jax.ShapeDtypeStruct((N, OUT_FEAT), f32),
        scratch_shapes=[
            pltpu.VMEM((N, OUT_FEAT), f32),
            pltpu.VMEM((N, N), bf16),
        ],
        compiler_params=pltpu.CompilerParams(
            dimension_semantics=("arbitrary",),
            vmem_limit_bytes=63 * 1024 * 1024,
        ),
    )(feat, adj, W1, W2)


def _decode_one(zb, z_all, wp, bp, wr, br, arec_ref, pred_ref, rec_ref):
    prod = lax.dot_general(zb, z_all, (((1,), (1,)), ((), ())),
                           preferred_element_type=jnp.float32)
    arec_ref[...] = jax.nn.sigmoid(prod)
    logits = jnp.dot(zb, wp, preferred_element_type=jnp.float32) + bp
    pred_ref[...] = jax.nn.softmax(logits, axis=-1)
    rec_ref[...] = jnp.dot(zb, wr, preferred_element_type=jnp.float32) + br


def _decode_kernel(zbs_ref, zbf_ref, zs_ref, zf_ref, wp_ref, bp_ref, wr_ref,
                   br_ref, arecs_ref, preds_ref, recs_ref,
                   arecf_ref, predf_ref, recf_ref):
    wp = wp_ref[...]
    bp = bp_ref[...]
    wr = wr_ref[...]
    br = br_ref[...]
    _decode_one(zbs_ref[...], zs_ref[...], wp, bp, wr, br,
                arecs_ref, preds_ref, recs_ref)
    _decode_one(zbf_ref[...], zf_ref[...], wp, bp, wr, br,
                arecf_ref, predf_ref, recf_ref)


def _blk(i):
    return (i, 0)


def _decode(z_s, z_f, Wp, bp2, Wr, br2):
    f32 = jnp.float32
    return pl.pallas_call(
        _decode_kernel,
        grid=(NB,),
        in_specs=[
            pl.BlockSpec((RB, OUT_FEAT), _blk),
            pl.BlockSpec((RB, OUT_FEAT), _blk),
            pl.BlockSpec((N, OUT_FEAT), _pin),
            pl.BlockSpec((N, OUT_FEAT), _pin),
            pl.BlockSpec((OUT_FEAT, CT), _pin),
            pl.BlockSpec((1, CT), _pin),
            pl.BlockSpec((OUT_FEAT, IN_FEAT), _pin),
            pl.BlockSpec((1, IN_FEAT), _pin),
        ],
        out_specs=[
            pl.BlockSpec((RB, N), _blk),
            pl.BlockSpec((RB, CT), _blk),
            pl.BlockSpec((RB, IN_FEAT), _blk),
            pl.BlockSpec((RB, N), _blk),
            pl.BlockSpec((RB, CT), _blk),
            pl.BlockSpec((RB, IN_FEAT), _blk),
        ],
        out_shape=[
            jax.ShapeDtypeStruct((N, N), f32),
            jax.ShapeDtypeStruct((N, CT), f32),
            jax.ShapeDtypeStruct((N, IN_FEAT), f32),
            jax.ShapeDtypeStruct((N, N), f32),
            jax.ShapeDtypeStruct((N, CT), f32),
            jax.ShapeDtypeStruct((N, IN_FEAT), f32),
        ],
        compiler_params=pltpu.CompilerParams(
            dimension_semantics=("arbitrary",),
            vmem_limit_bytes=63 * 1024 * 1024,
        ),
    )(z_s, z_f, z_s, z_f, Wp, bp2, Wr, br2)


def kernel(features, features_sc, adj_spatial, adj_feature, W1, W2, Wp, bp, Wr, br):
    bp2 = bp.reshape(1, CT)
    br2 = br.reshape(1, IN_FEAT)

    z_s = _encode(features, adj_spatial, W1, W2)
    z_f = _encode(features_sc, adj_feature, W1, W2)
    arec_s, pred_s, rec_s, arec_f, pred_f, rec_f = _decode(
        z_s, z_f, Wp, bp2, Wr, br2)

    return (z_s, z_f, rec_s, rec_f, arec_s, arec_f, pred_s, pred_f)


# speedup vs baseline: 1.1885x; 1.0062x over previous
import jax
import jax.numpy as jnp
from jax import lax
from jax.experimental import pallas as pl
from jax.experimental.pallas import tpu as pltpu

N = 4096
IN_FEAT = 512
HID_FEAT = 256
OUT_FEAT = 64
CT = 20

RB = 512
NB = N // RB

S_Z = NB
S_END = 2 * NB


def _encode_kernel(f_ref, a_ref, w1_ref, w2_ref, z_ref, q_scr, ab_scr):
    i = pl.program_id(0)

    @pl.when(i < S_Z)
    def _load():
        h = jnp.dot(f_ref[...], w1_ref[...], preferred_element_type=jnp.float32)
        g = jnp.dot(h, w2_ref[...], preferred_element_type=jnp.float32)
        blk = a_ref[...]
        p = jnp.dot(blk, g, preferred_element_type=jnp.float32)

        @pl.when(i == 0)
        def _init():
            q_scr[...] = p

        @pl.when(i > 0)
        def _acc():
            q_scr[...] += p

        ab_scr[:, pl.ds(i * RB, RB)] = blk.astype(jnp.bfloat16)

    @pl.when(i >= S_Z)
    def _z():
        k = i - S_Z
        z_ref[...] = jnp.dot(ab_scr[pl.ds(k * RB, RB), :],
                             q_scr[...].astype(jnp.bfloat16),
                             preferred_element_type=jnp.float32)


def _f_idx(i):
    return (jnp.minimum(i, NB - 1), 0)


def _a_idx(i):
    return (0, jnp.minimum(i, NB - 1))


def _z_idx(i):
    return (jnp.clip(i - S_Z, 0, NB - 1), 0)


def _pin(i):
    return (0, 0)


def _encode(feat, adj, W1, W2):
    f32 = jnp.float32
    bf16 = jnp.bfloat16
    return pl.pallas_call(
        _encode_kernel,
        grid=(S_END,),
        in_specs=[
            pl.BlockSpec((RB, IN_FEAT), _f_idx),
            pl.BlockSpec((N, RB), _a_idx),
            pl.BlockSpec((IN_FEAT, HID_FEAT), _pin),
            pl.BlockSpec((HID_FEAT, OUT_FEAT), _pin),
        ],
        out_specs=pl.BlockSpec((RB, OUT_FEAT), _z_idx),
        out_shape=jax.ShapeDtypeStruct((N, OUT_FEAT), f32),
        scratch_shapes=[
            pltpu.VMEM((N, OUT_FEAT), f32),
            pltpu.VMEM((N, N), bf16),
        ],
        compiler_params=pltpu.CompilerParams(
            dimension_semantics=("arbitrary",),
            vmem_limit_bytes=63 * 1024 * 1024,
        ),
    )(feat, adj, W1, W2)


def _decode_one(zb, z_all, wp, bp, wr, br, arec_ref, pred_ref, rec_ref):
    prod = lax.dot_general(zb.astype(jnp.bfloat16), z_all,
                           (((1,), (1,)), ((), ())),
                           preferred_element_type=jnp.float32)
    arec_ref[...] = jax.nn.sigmoid(prod)
    logits = jnp.dot(zb, wp, preferred_element_type=jnp.float32) + bp
    pred_ref[...] = jax.nn.softmax(logits, axis=-1)
    rec_ref[...] = jnp.dot(zb, wr, preferred_element_type=jnp.float32) + br


def _decode_kernel(zbs_ref, zbf_ref, zs_ref, zf_ref, wp_ref, bp_ref, wr_ref,
                   br_ref, arecs_ref, preds_ref, recs_ref,
                   arecf_ref, predf_ref, recf_ref):
    wp = wp_ref[...]
    bp = bp_ref[...]
    wr = wr_ref[...]
    br = br_ref[...]
    zs16 = zs_ref[...].astype(jnp.bfloat16)
    zf16 = zf_ref[...].astype(jnp.bfloat16)
    _decode_one(zbs_ref[...], zs16, wp, bp, wr, br,
                arecs_ref, preds_ref, recs_ref)
    _decode_one(zbf_ref[...], zf16, wp, bp, wr, br,
                arecf_ref, predf_ref, recf_ref)


def _blk(i):
    return (i, 0)


def _decode(z_s, z_f, Wp, bp2, Wr, br2):
    f32 = jnp.float32
    return pl.pallas_call(
        _decode_kernel,
        grid=(NB,),
        in_specs=[
            pl.BlockSpec((RB, OUT_FEAT), _blk),
            pl.BlockSpec((RB, OUT_FEAT), _blk),
            pl.BlockSpec((N, OUT_FEAT), _pin),
            pl.BlockSpec((N, OUT_FEAT), _pin),
            pl.BlockSpec((OUT_FEAT, CT), _pin),
            pl.BlockSpec((1, CT), _pin),
            pl.BlockSpec((OUT_FEAT, IN_FEAT), _pin),
            pl.BlockSpec((1, IN_FEAT), _pin),
        ],
        out_specs=[
            pl.BlockSpec((RB, N), _blk),
            pl.BlockSpec((RB, CT), _blk),
            pl.BlockSpec((RB, IN_FEAT), _blk),
            pl.BlockSpec((RB, N), _blk),
            pl.BlockSpec((RB, CT), _blk),
            pl.BlockSpec((RB, IN_FEAT), _blk),
        ],
        out_shape=[
            jax.ShapeDtypeStruct((N, N), f32),
            jax.ShapeDtypeStruct((N, CT), f32),
            jax.ShapeDtypeStruct((N, IN_FEAT), f32),
            jax.ShapeDtypeStruct((N, N), f32),
            jax.ShapeDtypeStruct((N, CT), f32),
            jax.ShapeDtypeStruct((N, IN_FEAT), f32),
        ],
        compiler_params=pltpu.CompilerParams(
            dimension_semantics=("arbitrary",),
            vmem_limit_bytes=63 * 1024 * 1024,
        ),
    )(z_s, z_f, z_s, z_f, Wp, bp2, Wr, br2)


def kernel(features, features_sc, adj_spatial, adj_feature, W1, W2, Wp, bp, Wr, br):
    bp2 = bp.reshape(1, CT)
    br2 = br.reshape(1, IN_FEAT)

    z_s = _encode(features, adj_spatial, W1, W2)
    z_f = _encode(features_sc, adj_feature, W1, W2)
    arec_s, pred_s, rec_s, arec_f, pred_f, rec_f = _decode(
        z_s, z_f, Wp, bp2, Wr, br2)

    return (z_s, z_f, rec_s, rec_f, arec_s, arec_f, pred_s, pred_f)


# speedup vs baseline: 1.1955x; 1.0059x over previous
import jax
import jax.numpy as jnp
from jax import lax
from jax.experimental import pallas as pl
from jax.experimental.pallas import tpu as pltpu

N = 4096
IN_FEAT = 512
HID_FEAT = 256
OUT_FEAT = 64
CT = 20

RB = 512
NB = N // RB

RZ = 1024
NZ = N // RZ

S_Z = NB
S_END = NB + NZ


def _encode_kernel(f_ref, a_ref, w1_ref, w2_ref, z_ref, q_scr, ab_scr):
    i = pl.program_id(0)

    @pl.when(i < S_Z)
    def _load():
        h = jnp.dot(f_ref[...], w1_ref[...], preferred_element_type=jnp.float32)
        g = jnp.dot(h, w2_ref[...], preferred_element_type=jnp.float32)
        blk = a_ref[...]
        p = jnp.dot(blk, g, preferred_element_type=jnp.float32)

        @pl.when(i == 0)
        def _init():
            q_scr[...] = p

        @pl.when(i > 0)
        def _acc():
            q_scr[...] += p

        ab_scr[:, pl.ds(i * RB, RB)] = blk.astype(jnp.bfloat16)

    @pl.when(i >= S_Z)
    def _z():
        k = i - S_Z
        z_ref[...] = jnp.dot(ab_scr[pl.ds(k * RZ, RZ), :],
                             q_scr[...].astype(jnp.bfloat16),
                             preferred_element_type=jnp.float32)


def _f_idx(i):
    return (jnp.minimum(i, NB - 1), 0)


def _a_idx(i):
    return (0, jnp.minimum(i, NB - 1))


def _z_idx(i):
    return (jnp.clip(i - S_Z, 0, NZ - 1), 0)


def _pin(i):
    return (0, 0)


def _encode(feat, adj, W1, W2):
    f32 = jnp.float32
    bf16 = jnp.bfloat16
    return pl.pallas_call(
        _encode_kernel,
        grid=(S_END,),
        in_specs=[
            pl.BlockSpec((RB, IN_FEAT), _f_idx),
            pl.BlockSpec((N, RB), _a_idx),
            pl.BlockSpec((IN_FEAT, HID_FEAT), _pin),
            pl.BlockSpec((HID_FEAT, OUT_FEAT), _pin),
        ],
        out_specs=pl.BlockSpec((RZ, OUT_FEAT), _z_idx),
        out_shape=jax.ShapeDtypeStruct((N, OUT_FEAT), f32),
        scratch_shapes=[
            pltpu.VMEM((N, OUT_FEAT), f32),
            pltpu.VMEM((N, N), bf16),
        ],
        compiler_params=pltpu.CompilerParams(
            dimension_semantics=("arbitrary",),
            vmem_limit_bytes=65340 * 1024,
        ),
    )(feat, adj, W1, W2)


def _decode_one(zb, z_all, wp, bp, wr, br, arec_ref, pred_ref, rec_ref):
    prod = lax.dot_general(zb.astype(jnp.bfloat16), z_all,
                           (((1,), (1,)), ((), ())),
                           preferred_element_type=jnp.float32)
    arec_ref[...] = jax.nn.sigmoid(prod)
    logits = jnp.dot(zb, wp, preferred_element_type=jnp.float32) + bp
    pred_ref[...] = jax.nn.softmax(logits, axis=-1)
    rec_ref[...] = jnp.dot(zb, wr, preferred_element_type=jnp.float32) + br


def _decode_kernel(zbs_ref, zbf_ref, zs_ref, zf_ref, wp_ref, bp_ref, wr_ref,
                   br_ref, arecs_ref, preds_ref, recs_ref,
                   arecf_ref, predf_ref, recf_ref):
    wp = wp_ref[...]
    bp = bp_ref[...]
    wr = wr_ref[...]
    br = br_ref[...]
    zs16 = zs_ref[...].astype(jnp.bfloat16)
    zf16 = zf_ref[...].astype(jnp.bfloat16)
    _decode_one(zbs_ref[...], zs16, wp, bp, wr, br,
                arecs_ref, preds_ref, recs_ref)
    _decode_one(zbf_ref[...], zf16, wp, bp, wr, br,
                arecf_ref, predf_ref, recf_ref)


def _blk(i):
    return (i, 0)


def _decode(z_s, z_f, Wp, bp2, Wr, br2):
    f32 = jnp.float32
    return pl.pallas_call(
        _decode_kernel,
        grid=(NB,),
        in_specs=[
            pl.BlockSpec((RB, OUT_FEAT), _blk),
            pl.BlockSpec((RB, OUT_FEAT), _blk),
            pl.BlockSpec((N, OUT_FEAT), _pin),
            pl.BlockSpec((N, OUT_FEAT), _pin),
            pl.BlockSpec((OUT_FEAT, CT), _pin),
            pl.BlockSpec((1, CT), _pin),
            pl.BlockSpec((OUT_FEAT, IN_FEAT), _pin),
            pl.BlockSpec((1, IN_FEAT), _pin),
        ],
        out_specs=[
            pl.BlockSpec((RB, N), _blk),
            pl.BlockSpec((RB, CT), _blk),
            pl.BlockSpec((RB, IN_FEAT), _blk),
            pl.BlockSpec((RB, N), _blk),
            pl.BlockSpec((RB, CT), _blk),
            pl.BlockSpec((RB, IN_FEAT), _blk),
        ],
        out_shape=[
            jax.ShapeDtypeStruct((N, N), f32),
            jax.ShapeDtypeStruct((N, CT), f32),
            jax.ShapeDtypeStruct((N, IN_FEAT), f32),
            jax.ShapeDtypeStruct((N, N), f32),
            jax.ShapeDtypeStruct((N, CT), f32),
            jax.ShapeDtypeStruct((N, IN_FEAT), f32),
        ],
        compiler_params=pltpu.CompilerParams(
            dimension_semantics=("arbitrary",),
            vmem_limit_bytes=65340 * 1024,
        ),
    )(z_s, z_f, z_s, z_f, Wp, bp2, Wr, br2)


def kernel(features, features_sc, adj_spatial, adj_feature, W1, W2, Wp, bp, Wr, br):
    bp2 = bp.reshape(1, CT)
    br2 = br.reshape(1, IN_FEAT)

    z_s = _encode(features, adj_spatial, W1, W2)
    z_f = _encode(features_sc, adj_feature, W1, W2)
    arec_s, pred_s, rec_s, arec_f, pred_f, rec_f = _decode(
        z_s, z_f, Wp, bp2, Wr, br2)

    return (z_s, z_f, rec_s, rec_f, arec_s, arec_f, pred_s, pred_f)
